# baseline (speedup 1.0000x reference)
import functools

import jax
import jax.numpy as jnp
import numpy as np
from jax.experimental import pallas as pl

B = 512
TV = 16
NC = 20
D = 1024
H = 128
DM = 512
NH = 2
DH = 256
FF = 2048
E = 16
K = 2
MH = 256
MO = 128
VSN = 1e-12

_INTERPRET = False


def _ln(x, g, b):
    m = x.mean(-1, keepdims=True)
    v = ((x - m) ** 2).mean(-1, keepdims=True)
    return (x - m) / jnp.sqrt(v + 1e-5) * g + b


_K1_BLK = 64


def _k1_body(video_ref, title_ref, author_ref, asr_ref,
             wv_ref, bv_ref, wt_ref, bt_ref, wa_ref, ba_ref, ws_ref, bs_ref,
             fea_ref, vf_ref):
    relu = jax.nn.relu
    vproj = relu(jnp.dot(video_ref[...], wv_ref[...],
                         preferred_element_type=jnp.float32) + bv_ref[...])
    vf = vproj.reshape(_K1_BLK, TV, H).mean(axis=1)
    tf = relu(jnp.dot(title_ref[...], wt_ref[...],
                      preferred_element_type=jnp.float32) + bt_ref[...])
    sf = relu(jnp.dot(asr_ref[...], ws_ref[...],
                      preferred_element_type=jnp.float32) + bs_ref[...])
    auf = relu(jnp.dot(author_ref[...], wa_ref[...],
                       preferred_element_type=jnp.float32) + ba_ref[...])
    fea_ref[...] = jnp.concatenate([vf, tf, sf, auf], axis=1)
    vf_ref[...] = vf


def _k1(video_flat, title, author, asr, p):
    nblk = B // _K1_BLK
    return pl.pallas_call(
        _k1_body,
        grid=(nblk,),
        in_specs=[
            pl.BlockSpec((_K1_BLK * TV, D), lambda i: (i, 0)),
            pl.BlockSpec((_K1_BLK, D), lambda i: (i, 0)),
            pl.BlockSpec((_K1_BLK, D), lambda i: (i, 0)),
            pl.BlockSpec((_K1_BLK, D), lambda i: (i, 0)),
            pl.BlockSpec((D, H), lambda i: (0, 0)),
            pl.BlockSpec((1, H), lambda i: (0, 0)),
            pl.BlockSpec((D, H), lambda i: (0, 0)),
            pl.BlockSpec((1, H), lambda i: (0, 0)),
            pl.BlockSpec((D, H), lambda i: (0, 0)),
            pl.BlockSpec((1, H), lambda i: (0, 0)),
            pl.BlockSpec((D, H), lambda i: (0, 0)),
            pl.BlockSpec((1, H), lambda i: (0, 0)),
        ],
        out_specs=[
            pl.BlockSpec((_K1_BLK, 4 * H), lambda i: (i, 0)),
            pl.BlockSpec((_K1_BLK, H), lambda i: (i, 0)),
        ],
        out_shape=[
            jax.ShapeDtypeStruct((B, 4 * H), jnp.float32),
            jax.ShapeDtypeStruct((B, H), jnp.float32),
        ],
        interpret=_INTERPRET,
    )(video_flat, title, author, asr,
      p['W_video'], p['b_video'].reshape(1, H),
      p['W_title'], p['b_title'].reshape(1, H),
      p['W_author'], p['b_author'].reshape(1, H),
      p['W_asr'], p['b_asr'].reshape(1, H))


def _k2_body(fea_ref, wq_ref, bq_ref, wk_ref, bk_ref, wv_ref, bv_ref,
             wo_ref, bo_ref, ln1g_ref, ln1b_ref, wff1_ref, bff1_ref,
             wff2_ref, bff2_ref, ln2g_ref, ln2b_ref, wg_ref,
             we1_ref, be1_ref, we2_ref, be2_ref,
             moefea_ref, moeloss_ref):
    relu = jax.nn.relu
    f32 = jnp.float32
    x = fea_ref[...]
    q = jnp.dot(x, wq_ref[...], preferred_element_type=f32) + bq_ref[...]
    k = jnp.dot(x, wk_ref[...], preferred_element_type=f32) + bk_ref[...]
    v = jnp.dot(x, wv_ref[...], preferred_element_type=f32) + bv_ref[...]
    scale = 1.0 / np.sqrt(DH)
    o_heads = []
    for h in range(NH):
        qh = q[:, h * DH:(h + 1) * DH]
        kh = k[:, h * DH:(h + 1) * DH]
        vh = v[:, h * DH:(h + 1) * DH]
        scores = jax.lax.dot_general(
            qh, kh, (((1,), (1,)), ((), ())),
            preferred_element_type=f32) * scale
        att = jax.nn.softmax(scores, axis=-1)
        o_heads.append(jnp.dot(att, vh, preferred_element_type=f32))
    o = jnp.dot(jnp.concatenate(o_heads, axis=1), wo_ref[...],
                preferred_element_type=f32) + bo_ref[...]
    x = _ln(x + o, ln1g_ref[...], ln1b_ref[...])
    ff = jnp.dot(relu(jnp.dot(x, wff1_ref[...], preferred_element_type=f32)
                      + bff1_ref[...]),
                 wff2_ref[...], preferred_element_type=f32) + bff2_ref[...]
    x = _ln(x + ff, ln2g_ref[...], ln2b_ref[...])

    logits = jnp.dot(x, wg_ref[...], preferred_element_type=f32)
    eio = jax.lax.broadcasted_iota(jnp.int32, (B, E), 1)
    m1 = jnp.max(logits, axis=1, keepdims=True)
    i1 = jnp.min(jnp.where(logits == m1, eio, E), axis=1, keepdims=True)
    masked = jnp.where(eio == i1, -jnp.inf, logits)
    m2 = jnp.max(masked, axis=1, keepdims=True)
    i2 = jnp.min(jnp.where(masked == m2, eio, E), axis=1, keepdims=True)
    t = jnp.exp(m2 - m1)
    w1 = 1.0 / (1.0 + t)
    w2 = t / (1.0 + t)

    acc = jnp.zeros((B, MO), dtype=f32)
    for e in range(E):
        ge = jnp.where(i1 == e, w1, 0.0) + jnp.where(i2 == e, w2, 0.0)
        he = relu(jnp.dot(x, we1_ref[e], preferred_element_type=f32)
                  + be1_ref[e:e + 1, :])
        ye = jnp.dot(he, we2_ref[e], preferred_element_type=f32) \
            + be2_ref[e:e + 1, :]
        acc = acc + ge * ye
    moefea_ref[...] = acc

    gates = (jnp.where(eio == i1, w1, 0.0) + jnp.where(eio == i2, w2, 0.0))
    imp = gates.sum(axis=0, keepdims=True)
    load = (gates > 0).astype(f32).sum(axis=0, keepdims=True)

    def cv(tv):
        m = tv.mean()
        var = ((tv - m) ** 2).mean()
        return var / (m * m + 1e-10)

    moeloss_ref[...] = jnp.reshape(cv(imp) + cv(load), (1, 1))


def _k2(fea, p):
    return pl.pallas_call(
        _k2_body,
        out_shape=[
            jax.ShapeDtypeStruct((B, MO), jnp.float32),
            jax.ShapeDtypeStruct((1, 1), jnp.float32),
        ],
        interpret=_INTERPRET,
    )(fea, p['Wq'], p['bq'].reshape(1, DM), p['Wk'], p['bk'].reshape(1, DM),
      p['Wv'], p['bv'].reshape(1, DM), p['Wo'], p['bo'].reshape(1, DM),
      p['ln1_g'].reshape(1, DM), p['ln1_b'].reshape(1, DM),
      p['Wff1'], p['bff1'].reshape(1, FF), p['Wff2'], p['bff2'].reshape(1, DM),
      p['ln2_g'].reshape(1, DM), p['ln2_b'].reshape(1, DM),
      p['Wg'], p['We1'], p['be1'], p['We2'], p['be2'])


_K3_P = 8


def _k3_body(cflat_ref, sen_ref, lens_ref, vf_ref,
             wc_ref, bc_ref, wx_ref, bx_ref, wp_ref,
             wg1_ref, bg1_ref, wg2_ref, bg2_ref,
             we1_ref, ben1_ref, we2_ref, ben2_ref, wo_ref, bo_ref,
             gx_ref, cg_ref, sm_ref, dg_ref, sp_ref):
    relu = jax.nn.relu
    f32 = jnp.float32
    pid = pl.program_id(0)

    cblk = cflat_ref[...]
    CF = relu(jnp.dot(cblk, wc_ref[...], preferred_element_type=f32)
              + bc_ref[...])
    CTX = relu(jnp.dot(cblk, wx_ref[...], preferred_element_type=f32)
               + bx_ref[...])
    p1 = wp_ref[0:1, :]
    p2 = wp_ref[1:2, :]
    r_iota = jax.lax.broadcasted_iota(jnp.int32, (NC, 1), 0)
    c_iota = jax.lax.broadcasted_iota(jnp.int32, (1, NC), 1)
    inv_s42 = 1.0 / np.sqrt(42.0)

    gx_rows = []
    pooled_rows = []
    sm_acc = jnp.zeros((), f32)
    dg_acc = jnp.zeros((), f32)
    sp_acc = jnp.zeros((), f32)
    for i in range(_K3_P):
        raw = cblk[i * NC:(i + 1) * NC, :]
        sen = sen_ref[i * NC:(i + 1) * NC, :]
        cf = CF[i * NC:(i + 1) * NC, :]
        ctx = CTX[i * NC:(i + 1) * NC, :]
        ln = lens_ref[i, 0]
        mask_r = (r_iota < ln).astype(f32)
        mask_c = (c_iota < ln).astype(f32)

        def pgram(pvec):
            w = raw * pvec
            n = jnp.sqrt((w * w).sum(axis=1, keepdims=True))
            wn = w / (n + 1e-8)
            return jax.lax.dot_general(wn, wn, (((1,), (1,)), ((), ())),
                                       preferred_element_type=f32)

        attn2 = (pgram(p1) + pgram(p2)) * 0.5
        adj = relu(attn2) * mask_r * mask_c
        rowsum_adj = adj.sum(axis=1, keepdims=True)
        cra = adj / (rowsum_adj + VSN)
        csa = 0.8 * sen + 0.2 * cra
        nv = relu(jnp.dot(jnp.dot(csa, ctx, preferred_element_type=f32),
                          we1_ref[...], preferred_element_type=f32)
                  + ben1_ref[...])
        outn = jnp.dot(nv, we2_ref[...], preferred_element_type=f32) \
            + ben2_ref[...]
        pooled_rows.append(jnp.max(outn + (mask_r - 1.0) * 1e9,
                                   axis=0, keepdims=True))

        hub = vf_ref[i:i + 1, :]
        ax_l = cf * 0.5 + hub * inv_s42
        ax_h = hub * (1.0 / 21.0) + cf.sum(axis=0, keepdims=True) * inv_s42
        g1l = relu(jnp.dot(ax_l, wg1_ref[...], preferred_element_type=f32)
                   + bg1_ref[...])
        g1h = relu(jnp.dot(ax_h, wg1_ref[...], preferred_element_type=f32)
                   + bg1_ref[...])
        a2l = g1l * 0.5 + g1h * inv_s42
        a2h = g1h * (1.0 / 21.0) + g1l.sum(axis=0, keepdims=True) * inv_s42
        g2l = jnp.dot(a2l, wg2_ref[...], preferred_element_type=f32) \
            + bg2_ref[...]
        g2h = jnp.dot(a2h, wg2_ref[...], preferred_element_type=f32) \
            + bg2_ref[...]
        gx_rows.append(jnp.maximum(jnp.max(g2l, axis=0, keepdims=True), g2h))

        G = jax.lax.dot_general(raw, raw, (((1,), (1,)), ((), ())),
                                preferred_element_type=f32)
        fn = (raw * raw).sum(axis=1, keepdims=True)
        rowsum_cra = cra.sum(axis=1, keepdims=True)
        sm_acc = sm_acc + (rowsum_cra * fn).sum() - (cra * G).sum()
        dg_acc = dg_acc + jnp.log(rowsum_cra + VSN).sum()
        sp_acc = sp_acc + (cra * cra).sum()

    pooled = jnp.concatenate(pooled_rows, axis=0)
    cg_ref[...] = jnp.dot(pooled, wo_ref[...], preferred_element_type=f32) \
        + bo_ref[...]
    gx_ref[...] = jnp.concatenate(gx_rows, axis=0)

    @pl.when(pid == 0)
    def _():
        sm_ref[...] = jnp.zeros((1, 1), f32)
        dg_ref[...] = jnp.zeros((1, 1), f32)
        sp_ref[...] = jnp.zeros((1, 1), f32)

    sm_ref[...] += jnp.reshape(sm_acc, (1, 1))
    dg_ref[...] += jnp.reshape(dg_acc, (1, 1))
    sp_ref[...] += jnp.reshape(sp_acc, (1, 1))


def _k3(cflat, senflat, lens2d, vf, p):
    nblk = B // _K3_P
    rows = _K3_P * NC
    return pl.pallas_call(
        _k3_body,
        grid=(nblk,),
        in_specs=[
            pl.BlockSpec((rows, D), lambda i: (i, 0)),
            pl.BlockSpec((rows, NC), lambda i: (i, 0)),
            pl.BlockSpec((_K3_P, 1), lambda i: (i, 0)),
            pl.BlockSpec((_K3_P, H), lambda i: (i, 0)),
            pl.BlockSpec((D, H), lambda i: (0, 0)),
            pl.BlockSpec((1, H), lambda i: (0, 0)),
            pl.BlockSpec((D, H), lambda i: (0, 0)),
            pl.BlockSpec((1, H), lambda i: (0, 0)),
            pl.BlockSpec((2, D), lambda i: (0, 0)),
            pl.BlockSpec((H, H), lambda i: (0, 0)),
            pl.BlockSpec((1, H), lambda i: (0, 0)),
            pl.BlockSpec((H, H), lambda i: (0, 0)),
            pl.BlockSpec((1, H), lambda i: (0, 0)),
            pl.BlockSpec((H, H), lambda i: (0, 0)),
            pl.BlockSpec((1, H), lambda i: (0, 0)),
            pl.BlockSpec((H, H), lambda i: (0, 0)),
            pl.BlockSpec((1, H), lambda i: (0, 0)),
            pl.BlockSpec((H, H), lambda i: (0, 0)),
            pl.BlockSpec((1, H), lambda i: (0, 0)),
        ],
        out_specs=[
            pl.BlockSpec((_K3_P, H), lambda i: (i, 0)),
            pl.BlockSpec((_K3_P, H), lambda i: (i, 0)),
            pl.BlockSpec((1, 1), lambda i: (0, 0)),
            pl.BlockSpec((1, 1), lambda i: (0, 0)),
            pl.BlockSpec((1, 1), lambda i: (0, 0)),
        ],
        out_shape=[
            jax.ShapeDtypeStruct((B, H), jnp.float32),
            jax.ShapeDtypeStruct((B, H), jnp.float32),
            jax.ShapeDtypeStruct((1, 1), jnp.float32),
            jax.ShapeDtypeStruct((1, 1), jnp.float32),
            jax.ShapeDtypeStruct((1, 1), jnp.float32),
        ],
        interpret=_INTERPRET,
    )(cflat, senflat, lens2d, vf,
      p['W_comment'], p['b_comment'].reshape(1, H),
      p['W_ctx'], p['b_ctx'].reshape(1, H),
      p['w_pers'],
      p['Wgnn1'], p['bgnn1'].reshape(1, H),
      p['Wgnn2'], p['bgnn2'].reshape(1, H),
      p['Wenc1'], p['benc1'].reshape(1, H),
      p['Wenc2'], p['benc2'].reshape(1, H),
      p['W_out'], p['b_out'].reshape(1, H))


def _k4_body(moefea_ref, gx_ref, cg_ref, wc1_ref, bc1_ref,
             lng_ref, lnb_ref, wc2_ref, bc2_ref,
             moeloss_ref, sm_ref, dg_ref, sp_ref,
             out_ref, loss_ref):
    f32 = jnp.float32
    feat = jnp.concatenate([moefea_ref[...], gx_ref[...], cg_ref[...]],
                           axis=1)
    h = jax.nn.relu(_ln(jnp.dot(feat, wc1_ref[...],
                                preferred_element_type=f32) + bc1_ref[...],
                        lng_ref[...], lnb_ref[...]))
    out_ref[...] = jnp.dot(h, wc2_ref[...], preferred_element_type=f32) \
        + bc2_ref[...]
    smooth = 0.2 * sm_ref[...] / (B * NC * NC)
    degree = -0.1 * dg_ref[...] / B / NC
    sparsity = 0.1 * sp_ref[...] / (B * NC * NC)
    loss_ref[...] = moeloss_ref[...] + smooth + degree + sparsity


def _k4(moe_fea, gx, cg, moe_loss, sm, dg, sp, p):
    HC = 3 * H // 2
    return pl.pallas_call(
        _k4_body,
        out_shape=[
            jax.ShapeDtypeStruct((B, 2), jnp.float32),
            jax.ShapeDtypeStruct((1, 1), jnp.float32),
        ],
        interpret=_INTERPRET,
    )(moe_fea, gx, cg, p['Wc1'], p['bc1'].reshape(1, HC),
      p['lnc_g'].reshape(1, HC), p['lnc_b'].reshape(1, HC),
      p['Wc2'], p['bc2'].reshape(1, 2),
      moe_loss, sm, dg, sp)


@jax.jit
def kernel(video_feas, title_feas, author_feas, asr_feas, comment_feas,
           comment_lens, sen_adj, params):
    return _run(video_feas, title_feas, author_feas, asr_feas,
                comment_feas, comment_lens, sen_adj, params)


def _run(video_feas, title_feas, author_feas, asr_feas, comment_feas,
         comment_lens, sen_adj, params):
    p = params
    video_flat = video_feas.reshape(B * TV, D)
    cflat = comment_feas.reshape(B * NC, D)
    senflat = sen_adj.reshape(B * NC, NC)
    lens2d = comment_lens.reshape(B, 1)

    fea, vf = _k1(video_flat, title_feas, author_feas, asr_feas, p)
    moe_fea, moe_loss = _k2(fea, p)
    gx, cg, sm, dg, sp = _k3(cflat, senflat, lens2d, vf, p)
    out, loss = _k4(moe_fea, gx, cg, moe_loss, sm, dg, sp, p)
    return out, loss[0, 0]


# speedup vs baseline: 1.6221x; 1.6221x over previous
import functools

import jax
import jax.numpy as jnp
import numpy as np
from jax.experimental import pallas as pl

B = 512
TV = 16
NC = 20
D = 1024
H = 128
DM = 512
NH = 2
DH = 256
FF = 2048
E = 16
K = 2
MH = 256
MO = 128
VSN = 1e-12

_INTERPRET = False


def _ln(x, g, b):
    m = x.mean(-1, keepdims=True)
    v = ((x - m) ** 2).mean(-1, keepdims=True)
    return (x - m) / jnp.sqrt(v + 1e-5) * g + b


_K1_BLK = 64


def _k1_body(video_ref, title_ref, author_ref, asr_ref,
             wv_ref, bv_ref, wt_ref, bt_ref, wa_ref, ba_ref, ws_ref, bs_ref,
             fea_ref, vf_ref):
    relu = jax.nn.relu
    vproj = relu(jnp.dot(video_ref[...], wv_ref[...],
                         preferred_element_type=jnp.float32) + bv_ref[...])
    vf = vproj.reshape(_K1_BLK, TV, H).mean(axis=1)
    tf = relu(jnp.dot(title_ref[...], wt_ref[...],
                      preferred_element_type=jnp.float32) + bt_ref[...])
    sf = relu(jnp.dot(asr_ref[...], ws_ref[...],
                      preferred_element_type=jnp.float32) + bs_ref[...])
    auf = relu(jnp.dot(author_ref[...], wa_ref[...],
                       preferred_element_type=jnp.float32) + ba_ref[...])
    fea_ref[...] = jnp.concatenate([vf, tf, sf, auf], axis=1)
    vf_ref[...] = vf


def _k1(video_flat, title, author, asr, p):
    nblk = B // _K1_BLK
    return pl.pallas_call(
        _k1_body,
        grid=(nblk,),
        in_specs=[
            pl.BlockSpec((_K1_BLK * TV, D), lambda i: (i, 0)),
            pl.BlockSpec((_K1_BLK, D), lambda i: (i, 0)),
            pl.BlockSpec((_K1_BLK, D), lambda i: (i, 0)),
            pl.BlockSpec((_K1_BLK, D), lambda i: (i, 0)),
            pl.BlockSpec((D, H), lambda i: (0, 0)),
            pl.BlockSpec((1, H), lambda i: (0, 0)),
            pl.BlockSpec((D, H), lambda i: (0, 0)),
            pl.BlockSpec((1, H), lambda i: (0, 0)),
            pl.BlockSpec((D, H), lambda i: (0, 0)),
            pl.BlockSpec((1, H), lambda i: (0, 0)),
            pl.BlockSpec((D, H), lambda i: (0, 0)),
            pl.BlockSpec((1, H), lambda i: (0, 0)),
        ],
        out_specs=[
            pl.BlockSpec((_K1_BLK, 4 * H), lambda i: (i, 0)),
            pl.BlockSpec((_K1_BLK, H), lambda i: (i, 0)),
        ],
        out_shape=[
            jax.ShapeDtypeStruct((B, 4 * H), jnp.float32),
            jax.ShapeDtypeStruct((B, H), jnp.float32),
        ],
        interpret=_INTERPRET,
    )(video_flat, title, author, asr,
      p['W_video'], p['b_video'].reshape(1, H),
      p['W_title'], p['b_title'].reshape(1, H),
      p['W_author'], p['b_author'].reshape(1, H),
      p['W_asr'], p['b_asr'].reshape(1, H))


def _k2_body(fea_ref, wq_ref, bq_ref, wk_ref, bk_ref, wv_ref, bv_ref,
             wo_ref, bo_ref, ln1g_ref, ln1b_ref, wff1_ref, bff1_ref,
             wff2_ref, bff2_ref, ln2g_ref, ln2b_ref, wg_ref,
             we1_ref, be1_ref, we2_ref, be2_ref,
             moefea_ref, moeloss_ref):
    relu = jax.nn.relu
    f32 = jnp.float32
    x = fea_ref[...]
    q = jnp.dot(x, wq_ref[...], preferred_element_type=f32) + bq_ref[...]
    k = jnp.dot(x, wk_ref[...], preferred_element_type=f32) + bk_ref[...]
    v = jnp.dot(x, wv_ref[...], preferred_element_type=f32) + bv_ref[...]
    scale = 1.0 / np.sqrt(DH)
    o_heads = []
    for h in range(NH):
        qh = q[:, h * DH:(h + 1) * DH]
        kh = k[:, h * DH:(h + 1) * DH]
        vh = v[:, h * DH:(h + 1) * DH]
        scores = jax.lax.dot_general(
            qh, kh, (((1,), (1,)), ((), ())),
            preferred_element_type=f32) * scale
        att = jax.nn.softmax(scores, axis=-1)
        o_heads.append(jnp.dot(att, vh, preferred_element_type=f32))
    o = jnp.dot(jnp.concatenate(o_heads, axis=1), wo_ref[...],
                preferred_element_type=f32) + bo_ref[...]
    x = _ln(x + o, ln1g_ref[...], ln1b_ref[...])
    ff = jnp.dot(relu(jnp.dot(x, wff1_ref[...], preferred_element_type=f32)
                      + bff1_ref[...]),
                 wff2_ref[...], preferred_element_type=f32) + bff2_ref[...]
    x = _ln(x + ff, ln2g_ref[...], ln2b_ref[...])

    logits = jnp.dot(x, wg_ref[...], preferred_element_type=f32)
    eio = jax.lax.broadcasted_iota(jnp.int32, (B, E), 1)
    m1 = jnp.max(logits, axis=1, keepdims=True)
    i1 = jnp.min(jnp.where(logits == m1, eio, E), axis=1, keepdims=True)
    masked = jnp.where(eio == i1, -jnp.inf, logits)
    m2 = jnp.max(masked, axis=1, keepdims=True)
    i2 = jnp.min(jnp.where(masked == m2, eio, E), axis=1, keepdims=True)
    t = jnp.exp(m2 - m1)
    w1 = 1.0 / (1.0 + t)
    w2 = t / (1.0 + t)

    acc = jnp.zeros((B, MO), dtype=f32)
    for e in range(E):
        ge = jnp.where(i1 == e, w1, 0.0) + jnp.where(i2 == e, w2, 0.0)
        he = relu(jnp.dot(x, we1_ref[e], preferred_element_type=f32)
                  + be1_ref[e:e + 1, :])
        ye = jnp.dot(he, we2_ref[e], preferred_element_type=f32) \
            + be2_ref[e:e + 1, :]
        acc = acc + ge * ye
    moefea_ref[...] = acc

    gates = (jnp.where(eio == i1, w1, 0.0) + jnp.where(eio == i2, w2, 0.0))
    imp = gates.sum(axis=0, keepdims=True)
    load = (gates > 0).astype(f32).sum(axis=0, keepdims=True)

    def cv(tv):
        m = tv.mean()
        var = ((tv - m) ** 2).mean()
        return var / (m * m + 1e-10)

    moeloss_ref[...] = jnp.reshape(cv(imp) + cv(load), (1, 1))


def _k2(fea, p):
    return pl.pallas_call(
        _k2_body,
        out_shape=[
            jax.ShapeDtypeStruct((B, MO), jnp.float32),
            jax.ShapeDtypeStruct((1, 1), jnp.float32),
        ],
        interpret=_INTERPRET,
    )(fea, p['Wq'], p['bq'].reshape(1, DM), p['Wk'], p['bk'].reshape(1, DM),
      p['Wv'], p['bv'].reshape(1, DM), p['Wo'], p['bo'].reshape(1, DM),
      p['ln1_g'].reshape(1, DM), p['ln1_b'].reshape(1, DM),
      p['Wff1'], p['bff1'].reshape(1, FF), p['Wff2'], p['bff2'].reshape(1, DM),
      p['ln2_g'].reshape(1, DM), p['ln2_b'].reshape(1, DM),
      p['Wg'], p['We1'], p['be1'], p['We2'], p['be2'])


_K3_P = 8


def _k3_body(cflat_ref, senbd_ref, lensr_ref, vf_ref,
             wc_ref, bc_ref, wx_ref, bx_ref, wp_ref,
             wg1_ref, bg1_ref, wg2_ref, bg2_ref,
             we1_ref, ben1_ref, we2_ref, ben2_ref, wo_ref, bo_ref,
             gx_ref, cg_ref, sm_ref, dg_ref, sp_ref):
    relu = jax.nn.relu
    f32 = jnp.float32
    pid = pl.program_id(0)
    R = _K3_P * NC
    inv_s42 = 1.0 / np.sqrt(42.0)

    raw = cflat_ref[...]
    CF = relu(jnp.dot(raw, wc_ref[...], preferred_element_type=f32)
              + bc_ref[...])
    CTX = relu(jnp.dot(raw, wx_ref[...], preferred_element_type=f32)
               + bx_ref[...])

    r_iota = jax.lax.broadcasted_iota(jnp.int32, (R, 1), 0)
    c_iota = jax.lax.broadcasted_iota(jnp.int32, (1, R), 1)
    rb = r_iota // NC
    cb = c_iota // NC
    mask_r = ((r_iota - rb * NC) < lensr_ref[...]).astype(f32)
    mask2d = jax.lax.dot_general(mask_r, mask_r, (((1,), (1,)), ((), ())),
                                 preferred_element_type=f32) \
        * (rb == cb).astype(f32)

    def pnorm(pvec):
        w = raw * pvec
        n = jnp.sqrt((w * w).sum(axis=1, keepdims=True))
        return w / (n + 1e-8)

    z = jnp.concatenate([pnorm(wp_ref[0:1, :]), pnorm(wp_ref[1:2, :])],
                        axis=1)
    attn2 = jax.lax.dot_general(z, z, (((1,), (1,)), ((), ())),
                                preferred_element_type=f32) * 0.5
    adj = relu(attn2) * mask2d
    rowsum_adj = adj.sum(axis=1, keepdims=True)
    cra = adj / (rowsum_adj + VSN)
    csa = 0.8 * senbd_ref[...] + 0.2 * cra
    nv = relu(jnp.dot(jnp.dot(csa, CTX, preferred_element_type=f32),
                      we1_ref[...], preferred_element_type=f32)
              + ben1_ref[...])
    outn = jnp.dot(nv, we2_ref[...], preferred_element_type=f32) \
        + ben2_ref[...]
    outn_m = outn + (mask_r - 1.0) * 1e9

    p_iota = jax.lax.broadcasted_iota(jnp.int32, (1, _K3_P), 1)
    S = (rb == p_iota).astype(f32)
    hub = vf_ref[...]
    hub_rows = jnp.dot(S, hub, preferred_element_type=f32)
    cf_sums = jax.lax.dot_general(S, CF, (((0,), (0,)), ((), ())),
                                  preferred_element_type=f32)
    ax_l = CF * 0.5 + hub_rows * inv_s42
    ax_h = hub * (1.0 / 21.0) + cf_sums * inv_s42
    g1l = relu(jnp.dot(ax_l, wg1_ref[...], preferred_element_type=f32)
               + bg1_ref[...])
    g1h = relu(jnp.dot(ax_h, wg1_ref[...], preferred_element_type=f32)
               + bg1_ref[...])
    g1l_sums = jax.lax.dot_general(S, g1l, (((0,), (0,)), ((), ())),
                                   preferred_element_type=f32)
    a2l = g1l * 0.5 + jnp.dot(S, g1h, preferred_element_type=f32) * inv_s42
    a2h = g1h * (1.0 / 21.0) + g1l_sums * inv_s42
    g2l = jnp.dot(a2l, wg2_ref[...], preferred_element_type=f32) \
        + bg2_ref[...]
    g2h = jnp.dot(a2h, wg2_ref[...], preferred_element_type=f32) \
        + bg2_ref[...]

    pooled_rows = []
    g2l_rows = []
    for i in range(_K3_P):
        pooled_rows.append(jnp.max(outn_m[i * NC:(i + 1) * NC, :],
                                   axis=0, keepdims=True))
        g2l_rows.append(jnp.max(g2l[i * NC:(i + 1) * NC, :],
                                axis=0, keepdims=True))
    pooled = jnp.concatenate(pooled_rows, axis=0)
    gx_ref[...] = jnp.maximum(jnp.concatenate(g2l_rows, axis=0), g2h)
    cg_ref[...] = jnp.dot(pooled, wo_ref[...], preferred_element_type=f32) \
        + bo_ref[...]

    G = jax.lax.dot_general(raw, raw, (((1,), (1,)), ((), ())),
                            preferred_element_type=f32)
    fn = (raw * raw).sum(axis=1, keepdims=True)
    rowsum_cra = cra.sum(axis=1, keepdims=True)
    sm_acc = (rowsum_cra * fn).sum() - (cra * G).sum()
    dg_acc = jnp.log(rowsum_cra + VSN).sum()
    sp_acc = (cra * cra).sum()

    @pl.when(pid == 0)
    def _():
        sm_ref[...] = jnp.zeros((1, 1), f32)
        dg_ref[...] = jnp.zeros((1, 1), f32)
        sp_ref[...] = jnp.zeros((1, 1), f32)

    sm_ref[...] += jnp.reshape(sm_acc, (1, 1))
    dg_ref[...] += jnp.reshape(dg_acc, (1, 1))
    sp_ref[...] += jnp.reshape(sp_acc, (1, 1))


def _k3(cflat, sen_bd, lens_r, vf, p):
    nblk = B // _K3_P
    rows = _K3_P * NC
    return pl.pallas_call(
        _k3_body,
        grid=(nblk,),
        in_specs=[
            pl.BlockSpec((rows, D), lambda i: (i, 0)),
            pl.BlockSpec((rows, rows), lambda i: (i, 0)),
            pl.BlockSpec((rows, 1), lambda i: (i, 0)),
            pl.BlockSpec((_K3_P, H), lambda i: (i, 0)),
            pl.BlockSpec((D, H), lambda i: (0, 0)),
            pl.BlockSpec((1, H), lambda i: (0, 0)),
            pl.BlockSpec((D, H), lambda i: (0, 0)),
            pl.BlockSpec((1, H), lambda i: (0, 0)),
            pl.BlockSpec((2, D), lambda i: (0, 0)),
            pl.BlockSpec((H, H), lambda i: (0, 0)),
            pl.BlockSpec((1, H), lambda i: (0, 0)),
            pl.BlockSpec((H, H), lambda i: (0, 0)),
            pl.BlockSpec((1, H), lambda i: (0, 0)),
            pl.BlockSpec((H, H), lambda i: (0, 0)),
            pl.BlockSpec((1, H), lambda i: (0, 0)),
            pl.BlockSpec((H, H), lambda i: (0, 0)),
            pl.BlockSpec((1, H), lambda i: (0, 0)),
            pl.BlockSpec((H, H), lambda i: (0, 0)),
            pl.BlockSpec((1, H), lambda i: (0, 0)),
        ],
        out_specs=[
            pl.BlockSpec((_K3_P, H), lambda i: (i, 0)),
            pl.BlockSpec((_K3_P, H), lambda i: (i, 0)),
            pl.BlockSpec((1, 1), lambda i: (0, 0)),
            pl.BlockSpec((1, 1), lambda i: (0, 0)),
            pl.BlockSpec((1, 1), lambda i: (0, 0)),
        ],
        out_shape=[
            jax.ShapeDtypeStruct((B, H), jnp.float32),
            jax.ShapeDtypeStruct((B, H), jnp.float32),
            jax.ShapeDtypeStruct((1, 1), jnp.float32),
            jax.ShapeDtypeStruct((1, 1), jnp.float32),
            jax.ShapeDtypeStruct((1, 1), jnp.float32),
        ],
        interpret=_INTERPRET,
    )(cflat, sen_bd, lens_r, vf,
      p['W_comment'], p['b_comment'].reshape(1, H),
      p['W_ctx'], p['b_ctx'].reshape(1, H),
      p['w_pers'],
      p['Wgnn1'], p['bgnn1'].reshape(1, H),
      p['Wgnn2'], p['bgnn2'].reshape(1, H),
      p['Wenc1'], p['benc1'].reshape(1, H),
      p['Wenc2'], p['benc2'].reshape(1, H),
      p['W_out'], p['b_out'].reshape(1, H))


def _k4_body(moefea_ref, gx_ref, cg_ref, wc1_ref, bc1_ref,
             lng_ref, lnb_ref, wc2_ref, bc2_ref,
             moeloss_ref, sm_ref, dg_ref, sp_ref,
             out_ref, loss_ref):
    f32 = jnp.float32
    feat = jnp.concatenate([moefea_ref[...], gx_ref[...], cg_ref[...]],
                           axis=1)
    h = jax.nn.relu(_ln(jnp.dot(feat, wc1_ref[...],
                                preferred_element_type=f32) + bc1_ref[...],
                        lng_ref[...], lnb_ref[...]))
    out_ref[...] = jnp.dot(h, wc2_ref[...], preferred_element_type=f32) \
        + bc2_ref[...]
    smooth = 0.2 * sm_ref[...] / (B * NC * NC)
    degree = -0.1 * dg_ref[...] / B / NC
    sparsity = 0.1 * sp_ref[...] / (B * NC * NC)
    loss_ref[...] = moeloss_ref[...] + smooth + degree + sparsity


def _k4(moe_fea, gx, cg, moe_loss, sm, dg, sp, p):
    HC = 3 * H // 2
    return pl.pallas_call(
        _k4_body,
        out_shape=[
            jax.ShapeDtypeStruct((B, 2), jnp.float32),
            jax.ShapeDtypeStruct((1, 1), jnp.float32),
        ],
        interpret=_INTERPRET,
    )(moe_fea, gx, cg, p['Wc1'], p['bc1'].reshape(1, HC),
      p['lnc_g'].reshape(1, HC), p['lnc_b'].reshape(1, HC),
      p['Wc2'], p['bc2'].reshape(1, 2),
      moe_loss, sm, dg, sp)


@jax.jit
def kernel(video_feas, title_feas, author_feas, asr_feas, comment_feas,
           comment_lens, sen_adj, params):
    return _run(video_feas, title_feas, author_feas, asr_feas,
                comment_feas, comment_lens, sen_adj, params)


def _run(video_feas, title_feas, author_feas, asr_feas, comment_feas,
         comment_lens, sen_adj, params):
    p = params
    video_flat = video_feas.reshape(B * TV, D)
    cflat = comment_feas.reshape(B * NC, D)
    rows = _K3_P * NC
    ngrp = B // _K3_P
    sen_bd = jnp.einsum(
        'gpij,pq->gpiqj',
        sen_adj.reshape(ngrp, _K3_P, NC, NC),
        jnp.eye(_K3_P, dtype=sen_adj.dtype)).reshape(B * NC, rows)
    lens_r = jnp.repeat(comment_lens, NC).reshape(B * NC, 1)

    fea, vf = _k1(video_flat, title_feas, author_feas, asr_feas, p)
    moe_fea, moe_loss = _k2(fea, p)
    gx, cg, sm, dg, sp = _k3(cflat, sen_bd, lens_r, vf, p)
    out, loss = _k4(moe_fea, gx, cg, moe_loss, sm, dg, sp, p)
    return out, loss[0, 0]


# speedup vs baseline: 1.6316x; 1.0059x over previous
import functools

import jax
import jax.numpy as jnp
import numpy as np
from jax.experimental import pallas as pl
from jax.experimental.pallas import tpu as pltpu

B = 512
TV = 16
NC = 20
D = 1024
H = 128
DM = 512
NH = 2
DH = 256
FF = 2048
E = 16
K = 2
MH = 256
MO = 128
VSN = 1e-12

_INTERPRET = False


def _ln(x, g, b):
    m = x.mean(-1, keepdims=True)
    v = ((x - m) ** 2).mean(-1, keepdims=True)
    return (x - m) / jnp.sqrt(v + 1e-5) * g + b


_K1_BLK = 64


def _k1_body(video_ref, title_ref, author_ref, asr_ref,
             wv_ref, bv_ref, wt_ref, bt_ref, wa_ref, ba_ref, ws_ref, bs_ref,
             fea_ref, vf_ref):
    relu = jax.nn.relu
    vproj = relu(jnp.dot(video_ref[...], wv_ref[...],
                         preferred_element_type=jnp.float32) + bv_ref[...])
    vf = vproj.reshape(_K1_BLK, TV, H).mean(axis=1)
    tf = relu(jnp.dot(title_ref[...], wt_ref[...],
                      preferred_element_type=jnp.float32) + bt_ref[...])
    sf = relu(jnp.dot(asr_ref[...], ws_ref[...],
                      preferred_element_type=jnp.float32) + bs_ref[...])
    auf = relu(jnp.dot(author_ref[...], wa_ref[...],
                       preferred_element_type=jnp.float32) + ba_ref[...])
    fea_ref[...] = jnp.concatenate([vf, tf, sf, auf], axis=1)
    vf_ref[...] = vf


def _k1(video_flat, title, author, asr, p):
    nblk = B // _K1_BLK
    return pl.pallas_call(
        _k1_body,
        grid=(nblk,),
        in_specs=[
            pl.BlockSpec((_K1_BLK * TV, D), lambda i: (i, 0)),
            pl.BlockSpec((_K1_BLK, D), lambda i: (i, 0)),
            pl.BlockSpec((_K1_BLK, D), lambda i: (i, 0)),
            pl.BlockSpec((_K1_BLK, D), lambda i: (i, 0)),
            pl.BlockSpec((D, H), lambda i: (0, 0)),
            pl.BlockSpec((1, H), lambda i: (0, 0)),
            pl.BlockSpec((D, H), lambda i: (0, 0)),
            pl.BlockSpec((1, H), lambda i: (0, 0)),
            pl.BlockSpec((D, H), lambda i: (0, 0)),
            pl.BlockSpec((1, H), lambda i: (0, 0)),
            pl.BlockSpec((D, H), lambda i: (0, 0)),
            pl.BlockSpec((1, H), lambda i: (0, 0)),
        ],
        out_specs=[
            pl.BlockSpec((_K1_BLK, 4 * H), lambda i: (i, 0)),
            pl.BlockSpec((_K1_BLK, H), lambda i: (i, 0)),
        ],
        out_shape=[
            jax.ShapeDtypeStruct((B, 4 * H), jnp.float32),
            jax.ShapeDtypeStruct((B, H), jnp.float32),
        ],
        compiler_params=pltpu.CompilerParams(
            dimension_semantics=("parallel",)),
        interpret=_INTERPRET,
    )(video_flat, title, author, asr,
      p['W_video'], p['b_video'].reshape(1, H),
      p['W_title'], p['b_title'].reshape(1, H),
      p['W_author'], p['b_author'].reshape(1, H),
      p['W_asr'], p['b_asr'].reshape(1, H))


def _k2_body(fea_ref, wq_ref, bq_ref, wk_ref, bk_ref, wv_ref, bv_ref,
             wo_ref, bo_ref, ln1g_ref, ln1b_ref, wff1_ref, bff1_ref,
             wff2_ref, bff2_ref, ln2g_ref, ln2b_ref, wg_ref,
             we1_ref, be1_ref, we2_ref, be2_ref,
             moefea_ref, moeloss_ref):
    relu = jax.nn.relu
    f32 = jnp.float32
    x = fea_ref[...]
    q = jnp.dot(x, wq_ref[...], preferred_element_type=f32) + bq_ref[...]
    k = jnp.dot(x, wk_ref[...], preferred_element_type=f32) + bk_ref[...]
    v = jnp.dot(x, wv_ref[...], preferred_element_type=f32) + bv_ref[...]
    scale = 1.0 / np.sqrt(DH)
    o_heads = []
    for h in range(NH):
        qh = q[:, h * DH:(h + 1) * DH]
        kh = k[:, h * DH:(h + 1) * DH]
        vh = v[:, h * DH:(h + 1) * DH]
        scores = jax.lax.dot_general(
            qh, kh, (((1,), (1,)), ((), ())),
            preferred_element_type=f32) * scale
        att = jax.nn.softmax(scores, axis=-1)
        o_heads.append(jnp.dot(att, vh, preferred_element_type=f32))
    o = jnp.dot(jnp.concatenate(o_heads, axis=1), wo_ref[...],
                preferred_element_type=f32) + bo_ref[...]
    x = _ln(x + o, ln1g_ref[...], ln1b_ref[...])
    ff = jnp.dot(relu(jnp.dot(x, wff1_ref[...], preferred_element_type=f32)
                      + bff1_ref[...]),
                 wff2_ref[...], preferred_element_type=f32) + bff2_ref[...]
    x = _ln(x + ff, ln2g_ref[...], ln2b_ref[...])

    logits = jnp.dot(x, wg_ref[...], preferred_element_type=f32)
    eio = jax.lax.broadcasted_iota(jnp.int32, (B, E), 1)
    m1 = jnp.max(logits, axis=1, keepdims=True)
    i1 = jnp.min(jnp.where(logits == m1, eio, E), axis=1, keepdims=True)
    masked = jnp.where(eio == i1, -jnp.inf, logits)
    m2 = jnp.max(masked, axis=1, keepdims=True)
    i2 = jnp.min(jnp.where(masked == m2, eio, E), axis=1, keepdims=True)
    t = jnp.exp(m2 - m1)
    w1 = 1.0 / (1.0 + t)
    w2 = t / (1.0 + t)

    acc = jnp.zeros((B, MO), dtype=f32)
    for e in range(E):
        ge = jnp.where(i1 == e, w1, 0.0) + jnp.where(i2 == e, w2, 0.0)
        he = relu(jnp.dot(x, we1_ref[e], preferred_element_type=f32)
                  + be1_ref[e:e + 1, :])
        ye = jnp.dot(he, we2_ref[e], preferred_element_type=f32) \
            + be2_ref[e:e + 1, :]
        acc = acc + ge * ye
    moefea_ref[...] = acc

    gates = (jnp.where(eio == i1, w1, 0.0) + jnp.where(eio == i2, w2, 0.0))
    imp = gates.sum(axis=0, keepdims=True)
    load = (gates > 0).astype(f32).sum(axis=0, keepdims=True)

    def cv(tv):
        m = tv.mean()
        var = ((tv - m) ** 2).mean()
        return var / (m * m + 1e-10)

    moeloss_ref[...] = jnp.reshape(cv(imp) + cv(load), (1, 1))


def _k2(fea, p):
    return pl.pallas_call(
        _k2_body,
        out_shape=[
            jax.ShapeDtypeStruct((B, MO), jnp.float32),
            jax.ShapeDtypeStruct((1, 1), jnp.float32),
        ],
        interpret=_INTERPRET,
    )(fea, p['Wq'], p['bq'].reshape(1, DM), p['Wk'], p['bk'].reshape(1, DM),
      p['Wv'], p['bv'].reshape(1, DM), p['Wo'], p['bo'].reshape(1, DM),
      p['ln1_g'].reshape(1, DM), p['ln1_b'].reshape(1, DM),
      p['Wff1'], p['bff1'].reshape(1, FF), p['Wff2'], p['bff2'].reshape(1, DM),
      p['ln2_g'].reshape(1, DM), p['ln2_b'].reshape(1, DM),
      p['Wg'], p['We1'], p['be1'], p['We2'], p['be2'])


_K3_P = 8


def _k3_body(cflat_ref, senbd_ref, lensr_ref, vf_ref,
             wc_ref, bc_ref, wx_ref, bx_ref, wp_ref,
             wg1_ref, bg1_ref, wg2_ref, bg2_ref,
             we1_ref, ben1_ref, we2_ref, ben2_ref, wo_ref, bo_ref,
             gx_ref, cg_ref, part_ref):
    relu = jax.nn.relu
    f32 = jnp.float32
    R = _K3_P * NC
    inv_s42 = 1.0 / np.sqrt(42.0)

    raw = cflat_ref[...]
    CF = relu(jnp.dot(raw, wc_ref[...], preferred_element_type=f32)
              + bc_ref[...])
    CTX = relu(jnp.dot(raw, wx_ref[...], preferred_element_type=f32)
               + bx_ref[...])

    r_iota = jax.lax.broadcasted_iota(jnp.int32, (R, 1), 0)
    c_iota = jax.lax.broadcasted_iota(jnp.int32, (1, R), 1)
    rb = r_iota // NC
    cb = c_iota // NC
    mask_r = ((r_iota - rb * NC) < lensr_ref[...]).astype(f32)
    mask2d = jax.lax.dot_general(mask_r, mask_r, (((1,), (1,)), ((), ())),
                                 preferred_element_type=f32) \
        * (rb == cb).astype(f32)

    def pnorm(pvec):
        w = raw * pvec
        n = jnp.sqrt((w * w).sum(axis=1, keepdims=True))
        return w / (n + 1e-8)

    z = jnp.concatenate([pnorm(wp_ref[0:1, :]), pnorm(wp_ref[1:2, :])],
                        axis=1)
    attn2 = jax.lax.dot_general(z, z, (((1,), (1,)), ((), ())),
                                preferred_element_type=f32) * 0.5
    adj = relu(attn2) * mask2d
    rowsum_adj = adj.sum(axis=1, keepdims=True)
    cra = adj / (rowsum_adj + VSN)
    csa = 0.8 * senbd_ref[...] + 0.2 * cra
    nv = relu(jnp.dot(jnp.dot(csa, CTX, preferred_element_type=f32),
                      we1_ref[...], preferred_element_type=f32)
              + ben1_ref[...])
    outn = jnp.dot(nv, we2_ref[...], preferred_element_type=f32) \
        + ben2_ref[...]
    outn_m = outn + (mask_r - 1.0) * 1e9

    p_iota = jax.lax.broadcasted_iota(jnp.int32, (1, _K3_P), 1)
    S = (rb == p_iota).astype(f32)
    hub = vf_ref[...]
    hub_rows = jnp.dot(S, hub, preferred_element_type=f32)
    cf_sums = jax.lax.dot_general(S, CF, (((0,), (0,)), ((), ())),
                                  preferred_element_type=f32)
    ax_l = CF * 0.5 + hub_rows * inv_s42
    ax_h = hub * (1.0 / 21.0) + cf_sums * inv_s42
    g1l = relu(jnp.dot(ax_l, wg1_ref[...], preferred_element_type=f32)
               + bg1_ref[...])
    g1h = relu(jnp.dot(ax_h, wg1_ref[...], preferred_element_type=f32)
               + bg1_ref[...])
    g1l_sums = jax.lax.dot_general(S, g1l, (((0,), (0,)), ((), ())),
                                   preferred_element_type=f32)
    a2l = g1l * 0.5 + jnp.dot(S, g1h, preferred_element_type=f32) * inv_s42
    a2h = g1h * (1.0 / 21.0) + g1l_sums * inv_s42
    g2l = jnp.dot(a2l, wg2_ref[...], preferred_element_type=f32) \
        + bg2_ref[...]
    g2h = jnp.dot(a2h, wg2_ref[...], preferred_element_type=f32) \
        + bg2_ref[...]

    pooled_rows = []
    g2l_rows = []
    for i in range(_K3_P):
        pooled_rows.append(jnp.max(outn_m[i * NC:(i + 1) * NC, :],
                                   axis=0, keepdims=True))
        g2l_rows.append(jnp.max(g2l[i * NC:(i + 1) * NC, :],
                                axis=0, keepdims=True))
    pooled = jnp.concatenate(pooled_rows, axis=0)
    gx_ref[...] = jnp.maximum(jnp.concatenate(g2l_rows, axis=0), g2h)
    cg_ref[...] = jnp.dot(pooled, wo_ref[...], preferred_element_type=f32) \
        + bo_ref[...]

    G = jax.lax.dot_general(raw, raw, (((1,), (1,)), ((), ())),
                            preferred_element_type=f32)
    fn = (raw * raw).sum(axis=1, keepdims=True)
    rowsum_cra = cra.sum(axis=1, keepdims=True)
    sm_acc = (rowsum_cra * fn).sum() - (cra * G).sum()
    dg_acc = jnp.log(rowsum_cra + VSN).sum()
    sp_acc = (cra * cra).sum()
    part_ref[...] = jnp.concatenate(
        [jnp.broadcast_to(jnp.reshape(sm_acc, (1, 1)), (1, H)),
         jnp.broadcast_to(jnp.reshape(dg_acc, (1, 1)), (1, H)),
         jnp.broadcast_to(jnp.reshape(sp_acc, (1, 1)), (1, H)),
         jnp.zeros((5, H), f32)], axis=0)


def _k3(cflat, sen_bd, lens_r, vf, p):
    nblk = B // _K3_P
    rows = _K3_P * NC
    return pl.pallas_call(
        _k3_body,
        grid=(nblk,),
        in_specs=[
            pl.BlockSpec((rows, D), lambda i: (i, 0)),
            pl.BlockSpec((rows, rows), lambda i: (i, 0)),
            pl.BlockSpec((rows, 1), lambda i: (i, 0)),
            pl.BlockSpec((_K3_P, H), lambda i: (i, 0)),
            pl.BlockSpec((D, H), lambda i: (0, 0)),
            pl.BlockSpec((1, H), lambda i: (0, 0)),
            pl.BlockSpec((D, H), lambda i: (0, 0)),
            pl.BlockSpec((1, H), lambda i: (0, 0)),
            pl.BlockSpec((2, D), lambda i: (0, 0)),
            pl.BlockSpec((H, H), lambda i: (0, 0)),
            pl.BlockSpec((1, H), lambda i: (0, 0)),
            pl.BlockSpec((H, H), lambda i: (0, 0)),
            pl.BlockSpec((1, H), lambda i: (0, 0)),
            pl.BlockSpec((H, H), lambda i: (0, 0)),
            pl.BlockSpec((1, H), lambda i: (0, 0)),
            pl.BlockSpec((H, H), lambda i: (0, 0)),
            pl.BlockSpec((1, H), lambda i: (0, 0)),
            pl.BlockSpec((H, H), lambda i: (0, 0)),
            pl.BlockSpec((1, H), lambda i: (0, 0)),
        ],
        out_specs=[
            pl.BlockSpec((_K3_P, H), lambda i: (i, 0)),
            pl.BlockSpec((_K3_P, H), lambda i: (i, 0)),
            pl.BlockSpec((8, H), lambda i: (i, 0)),
        ],
        out_shape=[
            jax.ShapeDtypeStruct((B, H), jnp.float32),
            jax.ShapeDtypeStruct((B, H), jnp.float32),
            jax.ShapeDtypeStruct((nblk * 8, H), jnp.float32),
        ],
        compiler_params=pltpu.CompilerParams(
            dimension_semantics=("parallel",)),
        interpret=_INTERPRET,
    )(cflat, sen_bd, lens_r, vf,
      p['W_comment'], p['b_comment'].reshape(1, H),
      p['W_ctx'], p['b_ctx'].reshape(1, H),
      p['w_pers'],
      p['Wgnn1'], p['bgnn1'].reshape(1, H),
      p['Wgnn2'], p['bgnn2'].reshape(1, H),
      p['Wenc1'], p['benc1'].reshape(1, H),
      p['Wenc2'], p['benc2'].reshape(1, H),
      p['W_out'], p['b_out'].reshape(1, H))


def _k4_body(moefea_ref, gx_ref, cg_ref, wc1_ref, bc1_ref,
             lng_ref, lnb_ref, wc2_ref, bc2_ref,
             moeloss_ref, part_ref,
             out_ref, loss_ref):
    f32 = jnp.float32
    feat = jnp.concatenate([moefea_ref[...], gx_ref[...], cg_ref[...]],
                           axis=1)
    h = jax.nn.relu(_ln(jnp.dot(feat, wc1_ref[...],
                                preferred_element_type=f32) + bc1_ref[...],
                        lng_ref[...], lnb_ref[...]))
    out_ref[...] = jnp.dot(h, wc2_ref[...], preferred_element_type=f32) \
        + bc2_ref[...]
    col0 = part_ref[...][:, 0:1]
    r8 = jax.lax.broadcasted_iota(jnp.int32, col0.shape, 0) % 8
    smooth = 0.2 * jnp.where(r8 == 0, col0, 0.0).sum() / (B * NC * NC)
    degree = -0.1 * jnp.where(r8 == 1, col0, 0.0).sum() / B / NC
    sparsity = 0.1 * jnp.where(r8 == 2, col0, 0.0).sum() / (B * NC * NC)
    loss_ref[...] = moeloss_ref[...] + jnp.reshape(
        smooth + degree + sparsity, (1, 1))


def _k4(moe_fea, gx, cg, moe_loss, part, p):
    HC = 3 * H // 2
    return pl.pallas_call(
        _k4_body,
        out_shape=[
            jax.ShapeDtypeStruct((B, 2), jnp.float32),
            jax.ShapeDtypeStruct((1, 1), jnp.float32),
        ],
        interpret=_INTERPRET,
    )(moe_fea, gx, cg, p['Wc1'], p['bc1'].reshape(1, HC),
      p['lnc_g'].reshape(1, HC), p['lnc_b'].reshape(1, HC),
      p['Wc2'], p['bc2'].reshape(1, 2),
      moe_loss, part)


@jax.jit
def kernel(video_feas, title_feas, author_feas, asr_feas, comment_feas,
           comment_lens, sen_adj, params):
    return _run(video_feas, title_feas, author_feas, asr_feas,
                comment_feas, comment_lens, sen_adj, params)


def _run(video_feas, title_feas, author_feas, asr_feas, comment_feas,
         comment_lens, sen_adj, params):
    p = params
    video_flat = video_feas.reshape(B * TV, D)
    cflat = comment_feas.reshape(B * NC, D)
    rows = _K3_P * NC
    ngrp = B // _K3_P
    sen_bd = jnp.einsum(
        'gpij,pq->gpiqj',
        sen_adj.reshape(ngrp, _K3_P, NC, NC),
        jnp.eye(_K3_P, dtype=sen_adj.dtype)).reshape(B * NC, rows)
    lens_r = jnp.repeat(comment_lens, NC).reshape(B * NC, 1)

    fea, vf = _k1(video_flat, title_feas, author_feas, asr_feas, p)
    moe_fea, moe_loss = _k2(fea, p)
    gx, cg, part = _k3(cflat, sen_bd, lens_r, vf, p)
    out, loss = _k4(moe_fea, gx, cg, moe_loss, part, p)
    return out, loss[0, 0]


# speedup vs baseline: 1.9270x; 1.1810x over previous
import functools

import jax
import jax.numpy as jnp
import numpy as np
from jax.experimental import pallas as pl
from jax.experimental.pallas import tpu as pltpu

B = 512
TV = 16
NC = 20
D = 1024
H = 128
DM = 512
NH = 2
DH = 256
FF = 2048
E = 16
K = 2
MH = 256
MO = 128
VSN = 1e-12

_INTERPRET = False


def _ln(x, g, b):
    m = x.mean(-1, keepdims=True)
    v = ((x - m) ** 2).mean(-1, keepdims=True)
    return (x - m) / jnp.sqrt(v + 1e-5) * g + b


_K1_BLK = 64


def _k1_body(video_ref, title_ref, author_ref, asr_ref,
             wv_ref, bv_ref, wt_ref, bt_ref, wa_ref, ba_ref, ws_ref, bs_ref,
             fea_ref, vf_ref):
    relu = jax.nn.relu
    vproj = relu(jnp.dot(video_ref[...], wv_ref[...],
                         preferred_element_type=jnp.float32) + bv_ref[...])
    vf = vproj.reshape(_K1_BLK, TV, H).mean(axis=1)
    tf = relu(jnp.dot(title_ref[...], wt_ref[...],
                      preferred_element_type=jnp.float32) + bt_ref[...])
    sf = relu(jnp.dot(asr_ref[...], ws_ref[...],
                      preferred_element_type=jnp.float32) + bs_ref[...])
    auf = relu(jnp.dot(author_ref[...], wa_ref[...],
                       preferred_element_type=jnp.float32) + ba_ref[...])
    fea_ref[...] = jnp.concatenate([vf, tf, sf, auf], axis=1)
    vf_ref[...] = vf


def _k1(video_flat, title, author, asr, p):
    nblk = B // _K1_BLK
    return pl.pallas_call(
        _k1_body,
        grid=(nblk,),
        in_specs=[
            pl.BlockSpec((_K1_BLK * TV, D), lambda i: (i, 0)),
            pl.BlockSpec((_K1_BLK, D), lambda i: (i, 0)),
            pl.BlockSpec((_K1_BLK, D), lambda i: (i, 0)),
            pl.BlockSpec((_K1_BLK, D), lambda i: (i, 0)),
            pl.BlockSpec((D, H), lambda i: (0, 0)),
            pl.BlockSpec((1, H), lambda i: (0, 0)),
            pl.BlockSpec((D, H), lambda i: (0, 0)),
            pl.BlockSpec((1, H), lambda i: (0, 0)),
            pl.BlockSpec((D, H), lambda i: (0, 0)),
            pl.BlockSpec((1, H), lambda i: (0, 0)),
            pl.BlockSpec((D, H), lambda i: (0, 0)),
            pl.BlockSpec((1, H), lambda i: (0, 0)),
        ],
        out_specs=[
            pl.BlockSpec((_K1_BLK, 4 * H), lambda i: (i, 0)),
            pl.BlockSpec((_K1_BLK, H), lambda i: (i, 0)),
        ],
        out_shape=[
            jax.ShapeDtypeStruct((B, 4 * H), jnp.float32),
            jax.ShapeDtypeStruct((B, H), jnp.float32),
        ],
        compiler_params=pltpu.CompilerParams(
            dimension_semantics=("parallel",)),
        interpret=_INTERPRET,
    )(video_flat, title, author, asr,
      p['W_video'], p['b_video'].reshape(1, H),
      p['W_title'], p['b_title'].reshape(1, H),
      p['W_author'], p['b_author'].reshape(1, H),
      p['W_asr'], p['b_asr'].reshape(1, H))


def _k2_body(fea_ref, wq_ref, bq_ref, wk_ref, bk_ref, wv_ref, bv_ref,
             wo_ref, bo_ref, ln1g_ref, ln1b_ref, wff1_ref, bff1_ref,
             wff2_ref, bff2_ref, ln2g_ref, ln2b_ref, wg_ref,
             we1_ref, be1_ref, we2_ref, be2_ref,
             moefea_ref, moeloss_ref):
    relu = jax.nn.relu
    f32 = jnp.float32
    x = fea_ref[...]
    q = jnp.dot(x, wq_ref[...], preferred_element_type=f32) + bq_ref[...]
    k = jnp.dot(x, wk_ref[...], preferred_element_type=f32) + bk_ref[...]
    v = jnp.dot(x, wv_ref[...], preferred_element_type=f32) + bv_ref[...]
    scale = 1.0 / np.sqrt(DH)
    o_heads = []
    for h in range(NH):
        qh = q[:, h * DH:(h + 1) * DH]
        kh = k[:, h * DH:(h + 1) * DH]
        vh = v[:, h * DH:(h + 1) * DH]
        scores = jax.lax.dot_general(
            qh, kh, (((1,), (1,)), ((), ())),
            preferred_element_type=f32) * scale
        att = jax.nn.softmax(scores, axis=-1)
        o_heads.append(jnp.dot(att, vh, preferred_element_type=f32))
    o = jnp.dot(jnp.concatenate(o_heads, axis=1), wo_ref[...],
                preferred_element_type=f32) + bo_ref[...]
    x = _ln(x + o, ln1g_ref[...], ln1b_ref[...])
    ff = jnp.dot(relu(jnp.dot(x, wff1_ref[...], preferred_element_type=f32)
                      + bff1_ref[...]),
                 wff2_ref[...], preferred_element_type=f32) + bff2_ref[...]
    x = _ln(x + ff, ln2g_ref[...], ln2b_ref[...])

    logits = jnp.dot(x, wg_ref[...], preferred_element_type=f32)
    eio = jax.lax.broadcasted_iota(jnp.int32, (B, E), 1)
    m1 = jnp.max(logits, axis=1, keepdims=True)
    i1 = jnp.min(jnp.where(logits == m1, eio, E), axis=1, keepdims=True)
    masked = jnp.where(eio == i1, -jnp.inf, logits)
    m2 = jnp.max(masked, axis=1, keepdims=True)
    i2 = jnp.min(jnp.where(masked == m2, eio, E), axis=1, keepdims=True)
    t = jnp.exp(m2 - m1)
    w1 = 1.0 / (1.0 + t)
    w2 = t / (1.0 + t)

    acc = jnp.zeros((B, MO), dtype=f32)
    for e in range(E):
        ge = jnp.where(i1 == e, w1, 0.0) + jnp.where(i2 == e, w2, 0.0)
        he = relu(jnp.dot(x, we1_ref[e], preferred_element_type=f32)
                  + be1_ref[e:e + 1, :])
        ye = jnp.dot(he, we2_ref[e], preferred_element_type=f32) \
            + be2_ref[e:e + 1, :]
        acc = acc + ge * ye
    moefea_ref[...] = acc

    gates = (jnp.where(eio == i1, w1, 0.0) + jnp.where(eio == i2, w2, 0.0))
    imp = gates.sum(axis=0, keepdims=True)
    load = (gates > 0).astype(f32).sum(axis=0, keepdims=True)

    def cv(tv):
        m = tv.mean()
        var = ((tv - m) ** 2).mean()
        return var / (m * m + 1e-10)

    moeloss_ref[...] = jnp.reshape(cv(imp) + cv(load), (1, 1))


def _k2(fea, p):
    return pl.pallas_call(
        _k2_body,
        out_shape=[
            jax.ShapeDtypeStruct((B, MO), jnp.float32),
            jax.ShapeDtypeStruct((1, 1), jnp.float32),
        ],
        interpret=_INTERPRET,
    )(fea, p['Wq'], p['bq'].reshape(1, DM), p['Wk'], p['bk'].reshape(1, DM),
      p['Wv'], p['bv'].reshape(1, DM), p['Wo'], p['bo'].reshape(1, DM),
      p['ln1_g'].reshape(1, DM), p['ln1_b'].reshape(1, DM),
      p['Wff1'], p['bff1'].reshape(1, FF), p['Wff2'], p['bff2'].reshape(1, DM),
      p['ln2_g'].reshape(1, DM), p['ln2_b'].reshape(1, DM),
      p['Wg'], p['We1'], p['be1'], p['We2'], p['be2'])


_K3_P = 8


def _k3_body(cflat_ref, senbd_ref, lensr_ref, vf_ref,
             wc_ref, bc_ref, wx_ref, bx_ref, wp_ref,
             wg1_ref, bg1_ref, wg2_ref, bg2_ref,
             we1_ref, ben1_ref, we2_ref, ben2_ref, wo_ref, bo_ref,
             gx_ref, cg_ref, part_ref):
    relu = jax.nn.relu
    f32 = jnp.float32
    R = _K3_P * NC
    inv_s42 = 1.0 / np.sqrt(42.0)

    raw = cflat_ref[...]
    CF = relu(jnp.dot(raw, wc_ref[...], preferred_element_type=f32)
              + bc_ref[...])
    CTX = relu(jnp.dot(raw, wx_ref[...], preferred_element_type=f32)
               + bx_ref[...])

    r_iota = jax.lax.broadcasted_iota(jnp.int32, (R, 1), 0)
    c_iota = jax.lax.broadcasted_iota(jnp.int32, (1, R), 1)
    rb = r_iota // NC
    cb = c_iota // NC
    eq_blk = (rb == cb).astype(f32)
    p_iota = jax.lax.broadcasted_iota(jnp.int32, (1, _K3_P), 1)
    S = (rb == p_iota).astype(f32)
    len_rows = jnp.dot(S, lensr_ref[...], preferred_element_type=f32)
    mask_r = ((r_iota - rb * NC).astype(f32) < len_rows).astype(f32)
    mask2d = jax.lax.dot_general(mask_r, mask_r, (((1,), (1,)), ((), ())),
                                 preferred_element_type=f32) * eq_blk

    def pnorm(pvec):
        w = raw * pvec
        n = jnp.sqrt((w * w).sum(axis=1, keepdims=True))
        return w / (n + 1e-8)

    z = jnp.concatenate([pnorm(wp_ref[0:1, :]), pnorm(wp_ref[1:2, :])],
                        axis=1)
    attn2 = jax.lax.dot_general(z, z, (((1,), (1,)), ((), ())),
                                preferred_element_type=f32) * 0.5
    adj = relu(attn2) * mask2d
    rowsum_adj = adj.sum(axis=1, keepdims=True)
    cra = adj / (rowsum_adj + VSN)
    sen_bd = jnp.concatenate([senbd_ref[...]] * _K3_P, axis=1) * eq_blk
    csa = 0.8 * sen_bd + 0.2 * cra
    nv = relu(jnp.dot(jnp.dot(csa, CTX, preferred_element_type=f32),
                      we1_ref[...], preferred_element_type=f32)
              + ben1_ref[...])
    outn = jnp.dot(nv, we2_ref[...], preferred_element_type=f32) \
        + ben2_ref[...]
    outn_m = outn + (mask_r - 1.0) * 1e9

    hub = vf_ref[...]
    hub_rows = jnp.dot(S, hub, preferred_element_type=f32)
    cf_sums = jax.lax.dot_general(S, CF, (((0,), (0,)), ((), ())),
                                  preferred_element_type=f32)
    ax_l = CF * 0.5 + hub_rows * inv_s42
    ax_h = hub * (1.0 / 21.0) + cf_sums * inv_s42
    g1l = relu(jnp.dot(ax_l, wg1_ref[...], preferred_element_type=f32)
               + bg1_ref[...])
    g1h = relu(jnp.dot(ax_h, wg1_ref[...], preferred_element_type=f32)
               + bg1_ref[...])
    g1l_sums = jax.lax.dot_general(S, g1l, (((0,), (0,)), ((), ())),
                                   preferred_element_type=f32)
    a2l = g1l * 0.5 + jnp.dot(S, g1h, preferred_element_type=f32) * inv_s42
    a2h = g1h * (1.0 / 21.0) + g1l_sums * inv_s42
    g2l = jnp.dot(a2l, wg2_ref[...], preferred_element_type=f32) \
        + bg2_ref[...]
    g2h = jnp.dot(a2h, wg2_ref[...], preferred_element_type=f32) \
        + bg2_ref[...]

    pooled_rows = []
    g2l_rows = []
    for i in range(_K3_P):
        pooled_rows.append(jnp.max(outn_m[i * NC:(i + 1) * NC, :],
                                   axis=0, keepdims=True))
        g2l_rows.append(jnp.max(g2l[i * NC:(i + 1) * NC, :],
                                axis=0, keepdims=True))
    pooled = jnp.concatenate(pooled_rows, axis=0)
    gx_ref[...] = jnp.maximum(jnp.concatenate(g2l_rows, axis=0), g2h)
    cg_ref[...] = jnp.dot(pooled, wo_ref[...], preferred_element_type=f32) \
        + bo_ref[...]

    G = jax.lax.dot_general(raw, raw, (((1,), (1,)), ((), ())),
                            preferred_element_type=f32)
    fn = (raw * raw).sum(axis=1, keepdims=True)
    rowsum_cra = cra.sum(axis=1, keepdims=True)
    sm_acc = (rowsum_cra * fn).sum() - (cra * G).sum()
    dg_acc = jnp.log(rowsum_cra + VSN).sum()
    sp_acc = (cra * cra).sum()
    part_ref[...] = jnp.concatenate(
        [jnp.broadcast_to(jnp.reshape(sm_acc, (1, 1)), (1, H)),
         jnp.broadcast_to(jnp.reshape(dg_acc, (1, 1)), (1, H)),
         jnp.broadcast_to(jnp.reshape(sp_acc, (1, 1)), (1, H)),
         jnp.zeros((5, H), f32)], axis=0)


def _k3(cflat, sen_bd, lens_r, vf, p):
    nblk = B // _K3_P
    rows = _K3_P * NC
    return pl.pallas_call(
        _k3_body,
        grid=(nblk,),
        in_specs=[
            pl.BlockSpec((rows, D), lambda i: (i, 0)),
            pl.BlockSpec((rows, NC), lambda i: (i, 0)),
            pl.BlockSpec((_K3_P, 1), lambda i: (i, 0)),
            pl.BlockSpec((_K3_P, H), lambda i: (i, 0)),
            pl.BlockSpec((D, H), lambda i: (0, 0)),
            pl.BlockSpec((1, H), lambda i: (0, 0)),
            pl.BlockSpec((D, H), lambda i: (0, 0)),
            pl.BlockSpec((1, H), lambda i: (0, 0)),
            pl.BlockSpec((2, D), lambda i: (0, 0)),
            pl.BlockSpec((H, H), lambda i: (0, 0)),
            pl.BlockSpec((1, H), lambda i: (0, 0)),
            pl.BlockSpec((H, H), lambda i: (0, 0)),
            pl.BlockSpec((1, H), lambda i: (0, 0)),
            pl.BlockSpec((H, H), lambda i: (0, 0)),
            pl.BlockSpec((1, H), lambda i: (0, 0)),
            pl.BlockSpec((H, H), lambda i: (0, 0)),
            pl.BlockSpec((1, H), lambda i: (0, 0)),
            pl.BlockSpec((H, H), lambda i: (0, 0)),
            pl.BlockSpec((1, H), lambda i: (0, 0)),
        ],
        out_specs=[
            pl.BlockSpec((_K3_P, H), lambda i: (i, 0)),
            pl.BlockSpec((_K3_P, H), lambda i: (i, 0)),
            pl.BlockSpec((8, H), lambda i: (i, 0)),
        ],
        out_shape=[
            jax.ShapeDtypeStruct((B, H), jnp.float32),
            jax.ShapeDtypeStruct((B, H), jnp.float32),
            jax.ShapeDtypeStruct((nblk * 8, H), jnp.float32),
        ],
        compiler_params=pltpu.CompilerParams(
            dimension_semantics=("parallel",)),
        interpret=_INTERPRET,
    )(cflat, sen_bd, lens_r, vf,
      p['W_comment'], p['b_comment'].reshape(1, H),
      p['W_ctx'], p['b_ctx'].reshape(1, H),
      p['w_pers'],
      p['Wgnn1'], p['bgnn1'].reshape(1, H),
      p['Wgnn2'], p['bgnn2'].reshape(1, H),
      p['Wenc1'], p['benc1'].reshape(1, H),
      p['Wenc2'], p['benc2'].reshape(1, H),
      p['W_out'], p['b_out'].reshape(1, H))


def _k4_body(moefea_ref, gx_ref, cg_ref, wc1_ref, bc1_ref,
             lng_ref, lnb_ref, wc2_ref, bc2_ref,
             moeloss_ref, part_ref,
             out_ref, loss_ref):
    f32 = jnp.float32
    feat = jnp.concatenate([moefea_ref[...], gx_ref[...], cg_ref[...]],
                           axis=1)
    h = jax.nn.relu(_ln(jnp.dot(feat, wc1_ref[...],
                                preferred_element_type=f32) + bc1_ref[...],
                        lng_ref[...], lnb_ref[...]))
    out_ref[...] = jnp.dot(h, wc2_ref[...], preferred_element_type=f32) \
        + bc2_ref[...]
    col0 = part_ref[...][:, 0:1]
    r8 = jax.lax.broadcasted_iota(jnp.int32, col0.shape, 0) % 8
    smooth = 0.2 * jnp.where(r8 == 0, col0, 0.0).sum() / (B * NC * NC)
    degree = -0.1 * jnp.where(r8 == 1, col0, 0.0).sum() / B / NC
    sparsity = 0.1 * jnp.where(r8 == 2, col0, 0.0).sum() / (B * NC * NC)
    loss_ref[...] = moeloss_ref[...] + jnp.reshape(
        smooth + degree + sparsity, (1, 1))


def _k4(moe_fea, gx, cg, moe_loss, part, p):
    HC = 3 * H // 2
    return pl.pallas_call(
        _k4_body,
        out_shape=[
            jax.ShapeDtypeStruct((B, 2), jnp.float32),
            jax.ShapeDtypeStruct((1, 1), jnp.float32),
        ],
        interpret=_INTERPRET,
    )(moe_fea, gx, cg, p['Wc1'], p['bc1'].reshape(1, HC),
      p['lnc_g'].reshape(1, HC), p['lnc_b'].reshape(1, HC),
      p['Wc2'], p['bc2'].reshape(1, 2),
      moe_loss, part)


@jax.jit
def kernel(video_feas, title_feas, author_feas, asr_feas, comment_feas,
           comment_lens, sen_adj, params):
    return _run(video_feas, title_feas, author_feas, asr_feas,
                comment_feas, comment_lens, sen_adj, params)


def _run(video_feas, title_feas, author_feas, asr_feas, comment_feas,
         comment_lens, sen_adj, params):
    p = params
    video_flat = video_feas.reshape(B * TV, D)
    cflat = comment_feas.reshape(B * NC, D)
    senflat = sen_adj.reshape(B * NC, NC)
    lens_f = comment_lens.astype(jnp.float32).reshape(B, 1)

    fea, vf = _k1(video_flat, title_feas, author_feas, asr_feas, p)
    moe_fea, moe_loss = _k2(fea, p)
    gx, cg, part = _k3(cflat, senflat, lens_f, vf, p)
    out, loss = _k4(moe_fea, gx, cg, moe_loss, part, p)
    return out, loss[0, 0]
